# Initial kernel scaffold; baseline (speedup 1.0000x reference)
import jax
import jax.numpy as jnp
from jax.experimental import pallas as pl


def kernel(pos, pos_dst, edge_index):
    raise NotImplementedError("write your pallas kernel here")


# speedup vs baseline: 34.3051x; 34.3051x over previous
import functools

import jax
import jax.numpy as jnp
from jax import lax
from jax.experimental import pallas as pl
from jax.experimental.pallas import tpu as pltpu
from jax.experimental.pallas import tpu_sc as plsc

N = 100000
E = 6400000
NPAD = 102400
NW = 32
EPW = E // NW
CHUNK = 2000
NCHUNK = EPW // CHUNK
VPC = CHUNK // 16

_NEG_INF = float("-inf")


def _sc_body(pos_t, ei, g, part, buf, idxbuf, valbuf):
    info = plsc.get_sparse_core_info()
    nc = info.num_cores
    wid = lax.axis_index("s") * nc + lax.axis_index("c")
    base0 = wid * EPW

    for d in range(3):
        pltpu.sync_copy(pos_t.at[pl.ds(d * N, N)], buf.at[pl.ds(0, N)])

        def gather_chunk(c, _):
            base = base0 + c * CHUNK
            pltpu.sync_copy(ei.at[pl.ds(base, CHUNK)], idxbuf)

            def gather_vreg(j, _):
                iv = idxbuf[pl.ds(j * 16, 16)]
                valbuf[pl.ds(j * 16, 16)] = plsc.load_gather(buf, [iv])
                return 0

            lax.fori_loop(0, VPC, gather_vreg, 0)
            pltpu.sync_copy(valbuf, g.at[pl.ds(d * E + base, CHUNK)])
            return 0

        lax.fori_loop(0, NCHUNK, gather_chunk, 0)

        def init_vreg(i, _):
            buf[pl.ds(i * 16, 16)] = jnp.full((16,), _NEG_INF, jnp.float32)
            return 0

        lax.fori_loop(0, NPAD // 16, init_vreg, 0)

        def scatter_chunk(c, _):
            base = base0 + c * CHUNK
            pltpu.sync_copy(ei.at[pl.ds(E + base, CHUNK)], idxbuf)
            pltpu.sync_copy(g.at[pl.ds(d * E + base, CHUNK)], valbuf)

            def rmw_vreg(j, _):
                cv = idxbuf[pl.ds(j * 16, 16)]
                vv = valbuf[pl.ds(j * 16, 16)]
                cur = plsc.load_gather(buf, [cv])
                plsc.store_scatter(buf, [cv], jnp.maximum(cur, vv))
                cur2 = plsc.load_gather(buf, [cv])
                plsc.store_scatter(buf, [cv], vv, mask=vv > cur2)
                cur3 = plsc.load_gather(buf, [cv])
                plsc.store_scatter(buf, [cv], vv, mask=vv > cur3)
                return 0

            lax.fori_loop(0, VPC, rmw_vreg, 0)
            return 0

        lax.fori_loop(0, NCHUNK, scatter_chunk, 0)

        pltpu.sync_copy(buf, part.at[pl.ds((d * NW + wid) * NPAD, NPAD)])


def _combine_body(part_ref, pd_ref, out_ref):
    x = part_ref[...]
    m = jnp.max(x, axis=1)
    out_ref[...] = m - pd_ref[...]


@jax.jit
def _run(pos, pos_dst, edge_index):
    pos_t = pos.T.reshape(3 * N)
    pd_t = jnp.pad(pos_dst.T.reshape(3, N), ((0, 0), (0, NPAD - N)))
    ei = edge_index.reshape(2 * E)

    mesh = plsc.VectorSubcoreMesh(core_axis_name="c", subcore_axis_name="s")
    sc = pl.kernel(
        _sc_body,
        mesh=mesh,
        out_type=(
            jax.ShapeDtypeStruct((3 * E,), jnp.float32),
            jax.ShapeDtypeStruct((3 * NW * NPAD,), jnp.float32),
        ),
        scratch_types=[
            pltpu.VMEM((NPAD,), jnp.float32),
            pltpu.VMEM((CHUNK,), jnp.int32),
            pltpu.VMEM((CHUNK,), jnp.float32),
        ],
        compiler_params=pltpu.CompilerParams(needs_layout_passes=False),
    )
    _, part = sc(pos_t, ei)
    part = part.reshape(3, NW, NPAD)

    BN = 12800
    res = pl.pallas_call(
        _combine_body,
        grid=(NPAD // BN,),
        in_specs=[
            pl.BlockSpec((3, NW, BN), lambda i: (0, 0, i)),
            pl.BlockSpec((3, BN), lambda i: (0, i)),
        ],
        out_specs=pl.BlockSpec((3, BN), lambda i: (0, i)),
        out_shape=jax.ShapeDtypeStruct((3, NPAD), jnp.float32),
    )(part, pd_t)
    return res[:, :N].T


def kernel(pos, pos_dst, edge_index):
    return _run(pos, pos_dst, edge_index)
